# Initial kernel scaffold; baseline (speedup 1.0000x reference)
import jax
import jax.numpy as jnp
from jax.experimental import pallas as pl


def kernel(x, ema_frequency_counter, W_enc, W_dec, pre_bias, latent_bias):
    raise NotImplementedError("write your pallas kernel here")


# speedup vs baseline: 9.0917x; 9.0917x over previous
import functools

import jax
import jax.numpy as jnp
from jax import lax
from jax.experimental import pallas as pl

_K = 32
_D = 768
_BLK = 256


def _fused_body(x_ref, we_ref, wd_ref, pb_ref, lb_ref, o_ref):
    pb = pb_ref[...]
    xc = x_ref[...] - pb
    acts = jnp.dot(xc, we_ref[...], preferred_element_type=jnp.float32)
    acts = acts + lb_ref[...]

    s = lax.bitcast_convert_type(acts, jnp.int32)
    v = jnp.where(s >= 0, s, s ^ jnp.int32(0x7FFFFFFF))

    rows = acts.shape[0]
    lo0 = jnp.full((rows, 1), jnp.iinfo(jnp.int32).min, jnp.int32)
    hi0 = jnp.full((rows, 1), jnp.iinfo(jnp.int32).max, jnp.int32)

    def step(_, carry):
        lo, hi = carry
        mid = (lo | hi) - ((lo ^ hi) >> 1)
        cnt = jnp.sum((v >= mid).astype(jnp.float32), axis=1, keepdims=True)
        ge = cnt >= float(_K)
        return jnp.where(ge, mid, lo), jnp.where(ge, hi, mid - 1)

    thr, _ = lax.fori_loop(0, 32, step, (lo0, hi0))

    keep = v >= thr
    acts_topk = jnp.where(keep, jnp.maximum(acts, 0.0), 0.0)
    out = jnp.dot(acts_topk, wd_ref[...], preferred_element_type=jnp.float32)
    o_ref[...] = out + pb


@jax.jit
def _run(x, w_enc, w_dec, pre_bias, latent_bias):
    n_tok, d = x.shape
    grid = (n_tok // _BLK,)
    return pl.pallas_call(
        _fused_body,
        grid=grid,
        in_specs=[
            pl.BlockSpec((_BLK, d), lambda i: (i, 0)),
            pl.BlockSpec((d, d), lambda i: (0, 0)),
            pl.BlockSpec((d, d), lambda i: (0, 0)),
            pl.BlockSpec((1, d), lambda i: (0, 0)),
            pl.BlockSpec((1, d), lambda i: (0, 0)),
        ],
        out_specs=pl.BlockSpec((_BLK, d), lambda i: (i, 0)),
        out_shape=jax.ShapeDtypeStruct((n_tok, d), jnp.float32),
    )(x, w_enc, w_dec, pre_bias.reshape(1, d), latent_bias.reshape(1, d))


def kernel(x, ema_frequency_counter, W_enc, W_dec, pre_bias, latent_bias):
    del ema_frequency_counter
    return _run(x, W_enc, W_dec, pre_bias, latent_bias)


# speedup vs baseline: 15.0013x; 1.6500x over previous
import functools

import jax
import jax.numpy as jnp
from jax import lax
from jax.experimental import pallas as pl

_K = 32
_D = 768
_BLK = 256


def _fused_body(x_ref, we_ref, wd_ref, pb_ref, lb_ref, o_ref):
    pb = pb_ref[...]
    xc = x_ref[...] - pb
    acts = jnp.dot(xc, we_ref[...], preferred_element_type=jnp.float32)
    acts = acts + lb_ref[...]

    s = lax.bitcast_convert_type(acts, jnp.int32)
    v = jnp.where(s >= 0, s, s ^ jnp.int32(0x7FFFFFFF))

    rows = acts.shape[0]
    lo = jnp.full((rows, 1), jnp.iinfo(jnp.int32).min, jnp.int32)
    hi = jnp.full((rows, 1), jnp.iinfo(jnp.int32).max, jnp.int32)

    for _ in range(32):
        mid = (lo | hi) - ((lo ^ hi) >> 1)
        cnt = jnp.sum((v >= mid).astype(jnp.float32), axis=1, keepdims=True)
        ge = cnt >= float(_K)
        lo = jnp.where(ge, mid, lo)
        hi = jnp.where(ge, hi, mid - 1)
    thr = lo

    keep = v >= thr
    acts_topk = jnp.where(keep, jnp.maximum(acts, 0.0), 0.0)
    out = jnp.dot(acts_topk.astype(jnp.bfloat16), wd_ref[...],
                  preferred_element_type=jnp.float32)
    o_ref[...] = out + pb


@jax.jit
def _run(x, w_enc, w_dec, pre_bias, latent_bias):
    n_tok, d = x.shape
    grid = (n_tok // _BLK,)
    return pl.pallas_call(
        _fused_body,
        grid=grid,
        in_specs=[
            pl.BlockSpec((_BLK, d), lambda i: (i, 0)),
            pl.BlockSpec((d, d), lambda i: (0, 0)),
            pl.BlockSpec((d, d), lambda i: (0, 0)),
            pl.BlockSpec((1, d), lambda i: (0, 0)),
            pl.BlockSpec((1, d), lambda i: (0, 0)),
        ],
        out_specs=pl.BlockSpec((_BLK, d), lambda i: (i, 0)),
        out_shape=jax.ShapeDtypeStruct((n_tok, d), jnp.float32),
    )(x, w_enc, w_dec.astype(jnp.bfloat16),
      pre_bias.reshape(1, d), latent_bias.reshape(1, d))


def kernel(x, ema_frequency_counter, W_enc, W_dec, pre_bias, latent_bias):
    del ema_frequency_counter
    return _run(x, W_enc, W_dec, pre_bias, latent_bias)


# speedup vs baseline: 15.1483x; 1.0098x over previous
import jax
import jax.numpy as jnp
from jax import lax
from jax.experimental import pallas as pl
from jax.experimental.pallas import tpu as pltpu

_K = 32
_D = 768
_BLK = 256


def _fused_body(x_ref, we_ref, wd_ref, pb_ref, lb_ref, o_ref, acts_ref):
    i = pl.program_id(0)
    nsteps = pl.num_programs(0)
    pb = pb_ref[...]
    slot = lax.rem(i, 2)

    @pl.when(i < nsteps - 1)
    def _encode():
        xc = x_ref[...] - pb
        acts = jnp.dot(xc, we_ref[...], preferred_element_type=jnp.float32)
        acts_ref[slot] = acts + lb_ref[...]

    @pl.when(i > 0)
    def _select_decode():
        acts = acts_ref[1 - slot]
        s = lax.bitcast_convert_type(acts, jnp.int32)
        v = jnp.where(s >= 0, s, s ^ jnp.int32(0x7FFFFFFF))

        rows = acts.shape[0]
        lo = jnp.full((rows, 1), jnp.iinfo(jnp.int32).min, jnp.int32)
        hi = jnp.full((rows, 1), jnp.iinfo(jnp.int32).max, jnp.int32)
        for _ in range(32):
            mid = (lo | hi) - ((lo ^ hi) >> 1)
            cnt = jnp.sum((v >= mid).astype(jnp.float32), axis=1, keepdims=True)
            ge = cnt >= float(_K)
            lo = jnp.where(ge, mid, lo)
            hi = jnp.where(ge, hi, mid - 1)

        keep = v >= lo
        acts_topk = jnp.where(keep, jnp.maximum(acts, 0.0), 0.0)
        out = jnp.dot(acts_topk.astype(jnp.bfloat16), wd_ref[...],
                      preferred_element_type=jnp.float32)
        o_ref[...] = out + pb


@jax.jit
def _run(x, w_enc, w_dec, pre_bias, latent_bias):
    n_tok, d = x.shape
    nblk = n_tok // _BLK
    grid = (nblk + 1,)
    return pl.pallas_call(
        _fused_body,
        grid=grid,
        in_specs=[
            pl.BlockSpec((_BLK, d), lambda i: (jnp.minimum(i, nblk - 1), 0)),
            pl.BlockSpec((d, d), lambda i: (0, 0)),
            pl.BlockSpec((d, d), lambda i: (0, 0)),
            pl.BlockSpec((1, d), lambda i: (0, 0)),
            pl.BlockSpec((1, d), lambda i: (0, 0)),
        ],
        out_specs=pl.BlockSpec((_BLK, d), lambda i: (jnp.maximum(i - 1, 0), 0)),
        out_shape=jax.ShapeDtypeStruct((n_tok, d), jnp.float32),
        scratch_shapes=[pltpu.VMEM((2, _BLK, d), jnp.float32)],
    )(x, w_enc, w_dec.astype(jnp.bfloat16),
      pre_bias.reshape(1, d), latent_bias.reshape(1, d))


def kernel(x, ema_frequency_counter, W_enc, W_dec, pre_bias, latent_bias):
    del ema_frequency_counter
    return _run(x, W_enc, W_dec, pre_bias, latent_bias)


# speedup vs baseline: 17.7889x; 1.1743x over previous
import functools

import jax
import jax.numpy as jnp
from jax import lax
from jax.experimental import pallas as pl
from jax.experimental.pallas import tpu as pltpu
from jax.experimental.pallas import tpu_sc as plsc

_K = 32
_D = 768
_BLK = 256
_NTOK = 16384
_NW = 32
_RPW = _NTOK // _NW
_CH = 16
_IL = 4


def _enc_body(x_ref, we_ref, pb_ref, lb_ref, o_ref):
    xc = x_ref[...] - pb_ref[...]
    acts = jnp.dot(xc, we_ref[...], preferred_element_type=jnp.float32)
    o_ref[...] = acts + lb_ref[...]


def _dec_body(a_ref, t_ref, wd_ref, pb_ref, o_ref):
    acts = a_ref[...]
    keep = acts >= t_ref[...]
    acts_topk = jnp.where(keep, jnp.maximum(acts, 0.0), 0.0)
    out = jnp.dot(acts_topk.astype(jnp.bfloat16), wd_ref[...],
                  preferred_element_type=jnp.float32)
    o_ref[...] = out + pb_ref[...]


def _rev(x):
    return lax.rev(x, (0,))


def _sorted32(a, b):
    rb = _rev(b)
    return jnp.sort(jnp.minimum(a, rb)), jnp.sort(jnp.maximum(a, rb))


def _top32_merge(t0, t1, w0, w1):
    p0 = jnp.maximum(t0, _rev(w1))
    p1 = jnp.maximum(t1, _rev(w0))
    return jnp.sort(jnp.minimum(p0, p1)), jnp.sort(jnp.maximum(p0, p1))


@functools.partial(
    pl.kernel,
    out_type=jax.ShapeDtypeStruct((_NTOK,), jnp.float32),
    mesh=plsc.VectorSubcoreMesh(core_axis_name="c", subcore_axis_name="s"),
    compiler_params=pltpu.CompilerParams(needs_layout_passes=False),
    scratch_types=[
        pltpu.VMEM((_CH, _D), jnp.float32),
        pltpu.VMEM((_RPW,), jnp.float32),
    ],
)
def _sc_thresholds(acts_hbm, thr_hbm, buf, thrbuf):
    wid = lax.axis_index("s") * 2 + lax.axis_index("c")
    base = wid * _RPW

    lane = lax.iota(jnp.int32, 16)

    def chunk_body(c, carry):
        pltpu.sync_copy(acts_hbm.at[pl.ds(base + c * _CH, _CH)], buf)
        thr_acc = jnp.zeros((16,), jnp.float32)
        for g in range(_CH // _IL):
            rows = [g * _IL + r for r in range(_IL)]
            st = []
            for r in rows:
                a = jnp.sort(buf[r, pl.ds(0, 16)])
                b = jnp.sort(buf[r, pl.ds(16, 16)])
                st.extend(_sorted32(a, b))

            def dbl_step(j, ts):
                nts = []
                for q, r in enumerate(rows):
                    v1 = jnp.sort(buf[r, pl.ds(32 + j * 32, 16)])
                    v2 = jnp.sort(buf[r, pl.ds(48 + j * 32, 16)])
                    w0, w1 = _sorted32(v1, v2)
                    nts.extend(_top32_merge(ts[2 * q], ts[2 * q + 1], w0, w1))
                return tuple(nts)

            st = lax.fori_loop(0, (_D - 32) // 32, dbl_step, tuple(st))
            for q, r in enumerate(rows):
                t = jnp.full((16,), jnp.min(st[2 * q]), jnp.float32)
                thr_acc = jnp.where(lane == r, t, thr_acc)
        thrbuf[pl.ds(c * _CH, _CH)] = thr_acc
        return carry

    lax.fori_loop(0, _RPW // _CH, chunk_body, 0)
    pltpu.sync_copy(thrbuf, thr_hbm.at[pl.ds(base, _RPW)])


@jax.jit
def _run(x, w_enc, w_dec, pre_bias, latent_bias):
    n_tok, d = x.shape
    pb = pre_bias.reshape(1, d)
    lb = latent_bias.reshape(1, d)
    nblk = n_tok // _BLK

    acts = pl.pallas_call(
        _enc_body,
        grid=(nblk,),
        in_specs=[
            pl.BlockSpec((_BLK, d), lambda i: (i, 0)),
            pl.BlockSpec((d, d), lambda i: (0, 0)),
            pl.BlockSpec((1, d), lambda i: (0, 0)),
            pl.BlockSpec((1, d), lambda i: (0, 0)),
        ],
        out_specs=pl.BlockSpec((_BLK, d), lambda i: (i, 0)),
        out_shape=jax.ShapeDtypeStruct((n_tok, d), jnp.float32),
    )(x, w_enc, pb, lb)

    thr = _sc_thresholds(acts)

    return pl.pallas_call(
        _dec_body,
        grid=(nblk,),
        in_specs=[
            pl.BlockSpec((_BLK, d), lambda i: (i, 0)),
            pl.BlockSpec((_BLK, 1), lambda i: (i, 0)),
            pl.BlockSpec((d, d), lambda i: (0, 0)),
            pl.BlockSpec((1, d), lambda i: (0, 0)),
        ],
        out_specs=pl.BlockSpec((_BLK, d), lambda i: (i, 0)),
        out_shape=jax.ShapeDtypeStruct((n_tok, d), jnp.float32),
    )(acts, thr.reshape(n_tok, 1), w_dec.astype(jnp.bfloat16), pb)


def kernel(x, ema_frequency_counter, W_enc, W_dec, pre_bias, latent_bias):
    del ema_frequency_counter
    return _run(x, W_enc, W_dec, pre_bias, latent_bias)


# speedup vs baseline: 18.2076x; 1.0235x over previous
import functools

import jax
import jax.numpy as jnp
from jax import lax
from jax.experimental import pallas as pl
from jax.experimental.pallas import tpu as pltpu
from jax.experimental.pallas import tpu_sc as plsc

_K = 32
_D = 768
_BLK = 256
_NTOK = 16384
_NW = 32
_RPW = _NTOK // _NW
_CH = 8
_IL = 8


def _enc_body(x_ref, we_ref, pb_ref, lb_ref, o_ref):
    xc = x_ref[...] - pb_ref[...]
    acts = jnp.dot(xc, we_ref[...], preferred_element_type=jnp.float32)
    o_ref[...] = acts + lb_ref[...]


def _dec_body(a_ref, t_ref, wd_ref, pb_ref, o_ref):
    acts = a_ref[...]
    keep = acts >= t_ref[...]
    acts_topk = jnp.where(keep, jnp.maximum(acts, 0.0), 0.0)
    out = jnp.dot(acts_topk.astype(jnp.bfloat16), wd_ref[...],
                  preferred_element_type=jnp.float32)
    o_ref[...] = out + pb_ref[...]


def _rev(x):
    return lax.rev(x, (0,))


def _sorted32(a, b):
    rb = _rev(b)
    return jnp.sort(jnp.minimum(a, rb)), jnp.sort(jnp.maximum(a, rb))


def _top32_merge(t0, t1, w0, w1):
    p0 = jnp.maximum(t0, _rev(w1))
    p1 = jnp.maximum(t1, _rev(w0))
    return jnp.sort(jnp.minimum(p0, p1)), jnp.sort(jnp.maximum(p0, p1))


@functools.partial(
    pl.kernel,
    out_type=jax.ShapeDtypeStruct((_NTOK,), jnp.float32),
    mesh=plsc.VectorSubcoreMesh(core_axis_name="c", subcore_axis_name="s"),
    compiler_params=pltpu.CompilerParams(needs_layout_passes=False),
    scratch_types=[
        pltpu.VMEM((_CH, _D), jnp.float32),
        pltpu.VMEM((_CH, _D), jnp.float32),
        pltpu.VMEM((_RPW,), jnp.float32),
        pltpu.SemaphoreType.DMA,
        pltpu.SemaphoreType.DMA,
    ],
)
def _sc_thresholds(acts_hbm, thr_hbm, buf_a, buf_b, thrbuf, sem_a, sem_b):
    wid = lax.axis_index("s") * 2 + lax.axis_index("c")
    base = wid * _RPW

    lane = lax.iota(jnp.int32, 16)

    def process(buf, thr_acc, lane_off):
        st = []
        for r in range(_IL):
            a = jnp.sort(buf[r, pl.ds(0, 16)])
            b = jnp.sort(buf[r, pl.ds(16, 16)])
            st.extend(_sorted32(a, b))

        def dbl_step(j, ts):
            nts = []
            for r in range(_IL):
                v1 = jnp.sort(buf[r, pl.ds(32 + j * 32, 16)])
                v2 = jnp.sort(buf[r, pl.ds(48 + j * 32, 16)])
                w0, w1 = _sorted32(v1, v2)
                nts.extend(_top32_merge(ts[2 * r], ts[2 * r + 1], w0, w1))
            return tuple(nts)

        st = lax.fori_loop(0, (_D - 32) // 32, dbl_step, tuple(st))
        for r in range(_IL):
            t = jnp.full((16,), jnp.min(st[2 * r]), jnp.float32)
            thr_acc = jnp.where(lane == lane_off + r, t, thr_acc)
        return thr_acc

    def pair_body(c, carry):
        row0 = base + c * 2 * _CH
        cp_a = pltpu.async_copy(acts_hbm.at[pl.ds(row0, _CH)], buf_a, sem_a)
        cp_b = pltpu.async_copy(acts_hbm.at[pl.ds(row0 + _CH, _CH)], buf_b, sem_b)
        thr_acc = jnp.zeros((16,), jnp.float32)
        cp_a.wait()
        thr_acc = process(buf_a, thr_acc, 0)
        cp_b.wait()
        thr_acc = process(buf_b, thr_acc, _CH)
        thrbuf[pl.ds(c * 2 * _CH, 2 * _CH)] = thr_acc
        return carry

    lax.fori_loop(0, _RPW // (2 * _CH), pair_body, 0)
    pltpu.sync_copy(thrbuf, thr_hbm.at[pl.ds(base, _RPW)])


@jax.jit
def _run(x, w_enc, w_dec, pre_bias, latent_bias):
    n_tok, d = x.shape
    pb = pre_bias.reshape(1, d)
    lb = latent_bias.reshape(1, d)
    nblk = n_tok // _BLK

    acts = pl.pallas_call(
        _enc_body,
        grid=(nblk,),
        in_specs=[
            pl.BlockSpec((_BLK, d), lambda i: (i, 0)),
            pl.BlockSpec((d, d), lambda i: (0, 0)),
            pl.BlockSpec((1, d), lambda i: (0, 0)),
            pl.BlockSpec((1, d), lambda i: (0, 0)),
        ],
        out_specs=pl.BlockSpec((_BLK, d), lambda i: (i, 0)),
        out_shape=jax.ShapeDtypeStruct((n_tok, d), jnp.float32),
    )(x, w_enc, pb, lb)

    thr = _sc_thresholds(acts)

    return pl.pallas_call(
        _dec_body,
        grid=(nblk,),
        in_specs=[
            pl.BlockSpec((_BLK, d), lambda i: (i, 0)),
            pl.BlockSpec((_BLK, 1), lambda i: (i, 0)),
            pl.BlockSpec((d, d), lambda i: (0, 0)),
            pl.BlockSpec((1, d), lambda i: (0, 0)),
        ],
        out_specs=pl.BlockSpec((_BLK, d), lambda i: (i, 0)),
        out_shape=jax.ShapeDtypeStruct((n_tok, d), jnp.float32),
    )(acts, thr.reshape(n_tok, 1), w_dec.astype(jnp.bfloat16), pb)


def kernel(x, ema_frequency_counter, W_enc, W_dec, pre_bias, latent_bias):
    del ema_frequency_counter
    return _run(x, W_enc, W_dec, pre_bias, latent_bias)
